# baseline (device time: 98654 ns/iter reference)
import jax
import jax.numpy as jnp
from jax import lax
from jax.experimental import pallas as pl
from jax.experimental.pallas import tpu as pltpu

N_DEV = 4
N_WCHUNK = 8


def kernel(x, w_mat, scale_x, scale_w):
    m_per, k = x.shape
    _, n = w_mat.shape
    n_per = n // N_DEV
    half = m_per // 2
    quart = m_per // 4
    wc = n_per // N_WCHUNK

    sx = scale_x.astype(jnp.float32)
    sw = scale_w.astype(jnp.float32)

    def body(x_hbm, w_hbm, sx_ref, sw_ref, out_ref, gather_ref, w8_ref,
             wstage_ref, ostage_ref, xstage_ref, wsems, osems, xsems,
             send_sems, recv_sems):
        my = lax.axis_index("i")
        left = (my + N_DEV - 1) % N_DEV
        right = (my + 1) % N_DEV
        opp = (my + 2) % N_DEV

        xdmas = [
            pltpu.make_async_copy(
                x_hbm.at[pl.ds(h * half, half), :], xstage_ref.at[h],
                xsems.at[h])
            for h in range(2)
        ]
        xdmas[0].start()
        xdmas[1].start()


        scale = sx_ref[0] * sw_ref[0]

        def rows(origin, lo, nrows):
            return pl.ds(origin * m_per + lo, nrows)

        def copy(sl, sem_i, target):
            return pltpu.make_async_remote_copy(
                src_ref=gather_ref.at[sl, :],
                dst_ref=gather_ref.at[sl, :],
                send_sem=send_sems.at[sem_i],
                recv_sem=recv_sems.at[sem_i],
                device_id=(target,),
                device_id_type=pl.DeviceIdType.MESH,
            )

        xdmas[0].wait()
        gather_ref[rows(my, 0, half), :] = xstage_ref[0].astype(
            jnp.float8_e4m3fn)
        sends = [copy(rows(my, 0, half), 0, right),
                 copy(rows(my, 0, half), 2, left)]
        sends[0].start()
        sends[1].start()
        xdmas[1].wait()
        gather_ref[rows(my, half, half), :] = xstage_ref[1].astype(
            jnp.float8_e4m3fn)
        sends += [copy(rows(my, half, half), 1, right),
                  copy(rows(my, half, half), 3, left)]
        sends[2].start()
        sends[3].start()

        col0 = my * n_per

        def wdma(c, buf):
            return pltpu.make_async_copy(
                w_hbm.at[:, pl.ds(col0 + c * wc, wc)],
                wstage_ref.at[buf],
                wsems.at[buf],
            )

        wdma(0, 0).start()
        for c in range(N_WCHUNK):
            if c + 1 < N_WCHUNK:
                wdma(c + 1, (c + 1) % 2).start()
            wdma(c, c % 2).wait()
            w8_ref[:, pl.ds(c * wc, wc)] = wstage_ref[c % 2].astype(
                jnp.float8_e4m3fn)

        p1 = 3 * m_per // 8
        p2 = half - p1
        copy(rows(left, 0, half), 0, right).wait_recv()
        sends += [copy(rows(left, 0, p1), 4, right),
                  copy(rows(left, p1, p2), 5, right)]
        sends[4].start()
        sends[5].start()
        copy(rows(right, 0, half), 2, left).wait_recv()
        sends += [copy(rows(right, half, p1), 6, left),
                  copy(rows(right, half + p1, p2), 7, left)]
        sends[6].start()
        sends[7].start()

        pending = [None, None]
        slot = 0

        def gemm_rows(origin, lo, nrows):
            nonlocal slot
            r = rows(origin, lo, nrows)
            a = gather_ref[r, :]
            acc = lax.dot_general(
                a, w8_ref[...],
                (((1,), (0,)), ((), ())),
                preferred_element_type=jnp.float32,
            )
            if pending[slot] is not None:
                pending[slot].wait()
            ostage_ref[slot, 0:nrows, :] = (acc * scale).astype(jnp.bfloat16)
            cp = pltpu.make_async_copy(
                ostage_ref.at[slot, pl.ds(0, nrows), :], out_ref.at[r, :],
                osems.at[slot])
            cp.start()
            pending[slot] = cp
            slot ^= 1

        for q in range(4):
            gemm_rows(my, q * quart, quart)
        for q in range(2):
            gemm_rows(left, q * quart, quart)
        for q in range(2):
            gemm_rows(right, q * quart, quart)
        copy(rows(left, half, half), 1, right).wait_recv()
        for q in range(2, 4):
            gemm_rows(left, q * quart, quart)
        copy(rows(right, half, half), 3, left).wait_recv()
        for q in range(2, 4):
            gemm_rows(right, q * quart, quart)
        copy(rows(opp, 0, p1), 4, right).wait_recv()
        gemm_rows(opp, 0, p1)
        copy(rows(opp, half, p1), 6, left).wait_recv()
        gemm_rows(opp, half, p1)
        copy(rows(opp, p1, p2), 5, right).wait_recv()
        gemm_rows(opp, p1, p2)
        copy(rows(opp, half + p1, p2), 7, left).wait_recv()
        gemm_rows(opp, half + p1, p2)

        for cp in pending:
            if cp is not None:
                cp.wait()
        for s in sends:
            s.wait_send()

    return pl.pallas_call(
        body,
        out_shape=jax.ShapeDtypeStruct((N_DEV * m_per, n_per), jnp.bfloat16),
        in_specs=[
            pl.BlockSpec(memory_space=pltpu.MemorySpace.HBM),
            pl.BlockSpec(memory_space=pltpu.MemorySpace.HBM),
            pl.BlockSpec(memory_space=pltpu.SMEM),
            pl.BlockSpec(memory_space=pltpu.SMEM),
        ],
        out_specs=pl.BlockSpec(memory_space=pltpu.MemorySpace.HBM),
        scratch_shapes=[
            pltpu.VMEM((N_DEV * m_per, k), jnp.float8_e4m3fn),
            pltpu.VMEM((k, n_per), jnp.float8_e4m3fn),
            pltpu.VMEM((2, k, n_per // N_WCHUNK), jnp.float32),
            pltpu.VMEM((2, 3 * m_per // 8, n_per), jnp.bfloat16),
            pltpu.VMEM((2, m_per // 2, k), jnp.float32),
            pltpu.SemaphoreType.DMA((2,)),
            pltpu.SemaphoreType.DMA((2,)),
            pltpu.SemaphoreType.DMA((2,)),
            pltpu.SemaphoreType.DMA((8,)),
            pltpu.SemaphoreType.DMA((8,)),
        ],
        compiler_params=pltpu.CompilerParams(
            vmem_limit_bytes=60 * 1024 * 1024,
        ),
    )(x, w_mat, sx, sw)


# device time: 93595 ns/iter; 1.0541x vs baseline; 1.0541x over previous
import jax
import jax.numpy as jnp
from jax import lax
from jax.experimental import pallas as pl
from jax.experimental.pallas import tpu as pltpu

N_DEV = 4
N_WCHUNK = 8


def kernel(x, w_mat, scale_x, scale_w):
    m_per, k = x.shape
    _, n = w_mat.shape
    n_per = n // N_DEV
    half = m_per // 2
    quart = m_per // 4
    wc = n_per // N_WCHUNK

    sx = scale_x.astype(jnp.float32)
    sw = scale_w.astype(jnp.float32)

    def body(x_hbm, w_hbm, sx_ref, sw_ref, out_ref, gather_ref, w8_ref,
             wstage_ref, ostage_ref, xstage_ref, wsems, osems, xsems,
             send_sems, recv_sems):
        my = lax.axis_index("i")
        left = (my + N_DEV - 1) % N_DEV
        right = (my + 1) % N_DEV
        opp = (my + 2) % N_DEV

        quart_l = m_per // 4

        def xdma(q):
            return pltpu.make_async_copy(
                x_hbm.at[pl.ds(q * quart_l, quart_l), :],
                xstage_ref.at[q % 2], xsems.at[q % 2])

        xdma(0).start()
        xdma(1).start()

        barrier_sem = pltpu.get_barrier_semaphore()
        for nbr in (left, right):
            pl.semaphore_signal(
                barrier_sem, inc=1,
                device_id=(nbr,), device_id_type=pl.DeviceIdType.MESH,
            )
        pl.semaphore_wait(barrier_sem, 2)

        scale = sx_ref[0] * sw_ref[0]

        def rows(origin, lo, nrows):
            return pl.ds(origin * m_per + lo, nrows)

        def copy(sl, sem_i, target):
            return pltpu.make_async_remote_copy(
                src_ref=gather_ref.at[sl, :],
                dst_ref=gather_ref.at[sl, :],
                send_sem=send_sems.at[sem_i],
                recv_sem=recv_sems.at[sem_i],
                device_id=(target,),
                device_id_type=pl.DeviceIdType.MESH,
            )

        sends = []
        for q in range(4):
            xdma(q).wait()
            gather_ref[rows(my, q * quart, quart), :] = (
                xstage_ref[q % 2].astype(jnp.float8_e4m3fn))
            if q + 2 < 4:
                xdma(q + 2).start()
            s_r = copy(rows(my, q * quart, quart), q, right)
            s_l = copy(rows(my, q * quart, quart), 4 + q, left)
            s_r.start()
            s_l.start()
            sends += [s_r, s_l]

        col0 = my * n_per

        def wdma(c, buf):
            return pltpu.make_async_copy(
                w_hbm.at[:, pl.ds(col0 + c * wc, wc)],
                wstage_ref.at[buf],
                wsems.at[buf],
            )

        wdma(0, 0).start()
        for c in range(N_WCHUNK):
            if c + 1 < N_WCHUNK:
                wdma(c + 1, (c + 1) % 2).start()
            wdma(c, c % 2).wait()
            w8_ref[:, pl.ds(c * wc, wc)] = wstage_ref[c % 2].astype(
                jnp.float8_e4m3fn)

        p1 = 3 * m_per // 8
        p2 = half - p1
        copy(rows(left, 0, quart), 0, right).wait_recv()
        copy(rows(left, quart, quart), 1, right).wait_recv()
        fwd_r = [copy(rows(left, 0, p1), 8, right),
                 copy(rows(left, p1, p2), 9, right)]
        fwd_r[0].start()
        fwd_r[1].start()
        sends += fwd_r
        copy(rows(right, 0, quart), 4, left).wait_recv()
        copy(rows(right, quart, quart), 5, left).wait_recv()
        fwd_l = [copy(rows(right, half, p1), 10, left),
                 copy(rows(right, half + p1, p2), 11, left)]
        fwd_l[0].start()
        fwd_l[1].start()
        sends += fwd_l

        pending = [None, None]
        slot = 0

        def gemm_rows(origin, lo, nrows):
            nonlocal slot
            r = rows(origin, lo, nrows)
            a = gather_ref[r, :]
            acc = lax.dot_general(
                a, w8_ref[...],
                (((1,), (0,)), ((), ())),
                preferred_element_type=jnp.float32,
            )
            if pending[slot] is not None:
                pending[slot].wait()
            ostage_ref[slot, 0:nrows, :] = (acc * scale).astype(jnp.bfloat16)
            cp = pltpu.make_async_copy(
                ostage_ref.at[slot, pl.ds(0, nrows), :], out_ref.at[r, :],
                osems.at[slot])
            cp.start()
            pending[slot] = cp
            slot ^= 1

        for q in range(4):
            gemm_rows(my, q * quart, quart)
        for q in range(2):
            gemm_rows(left, q * quart, quart)
        for q in range(2):
            gemm_rows(right, q * quart, quart)
        copy(rows(left, half, quart), 2, right).wait_recv()
        gemm_rows(left, half, quart)
        copy(rows(right, half, quart), 6, left).wait_recv()
        gemm_rows(right, half, quart)
        copy(rows(left, 3 * quart, quart), 3, right).wait_recv()
        gemm_rows(left, 3 * quart, quart)
        copy(rows(right, 3 * quart, quart), 7, left).wait_recv()
        gemm_rows(right, 3 * quart, quart)
        copy(rows(opp, 0, p1), 8, right).wait_recv()
        gemm_rows(opp, 0, p1)
        copy(rows(opp, half, p1), 10, left).wait_recv()
        gemm_rows(opp, half, p1)
        copy(rows(opp, p1, p2), 9, right).wait_recv()
        gemm_rows(opp, p1, p2)
        copy(rows(opp, half + p1, p2), 11, left).wait_recv()
        gemm_rows(opp, half + p1, p2)

        for cp in pending:
            if cp is not None:
                cp.wait()
        for s in sends:
            s.wait_send()

    return pl.pallas_call(
        body,
        out_shape=jax.ShapeDtypeStruct((N_DEV * m_per, n_per), jnp.bfloat16),
        in_specs=[
            pl.BlockSpec(memory_space=pltpu.MemorySpace.HBM),
            pl.BlockSpec(memory_space=pltpu.MemorySpace.HBM),
            pl.BlockSpec(memory_space=pltpu.SMEM),
            pl.BlockSpec(memory_space=pltpu.SMEM),
        ],
        out_specs=pl.BlockSpec(memory_space=pltpu.MemorySpace.HBM),
        scratch_shapes=[
            pltpu.VMEM((N_DEV * m_per, k), jnp.float8_e4m3fn),
            pltpu.VMEM((k, n_per), jnp.float8_e4m3fn),
            pltpu.VMEM((2, k, n_per // N_WCHUNK), jnp.float32),
            pltpu.VMEM((2, 3 * m_per // 8, n_per), jnp.bfloat16),
            pltpu.VMEM((2, m_per // 4, k), jnp.float32),
            pltpu.SemaphoreType.DMA((2,)),
            pltpu.SemaphoreType.DMA((2,)),
            pltpu.SemaphoreType.DMA((2,)),
            pltpu.SemaphoreType.DMA((12,)),
            pltpu.SemaphoreType.DMA((12,)),
        ],
        compiler_params=pltpu.CompilerParams(
            collective_id=0,
            vmem_limit_bytes=60 * 1024 * 1024,
        ),
    )(x, w_mat, sx, sw)
